# baseline (speedup 1.0000x reference)
import functools

import jax
import jax.numpy as jnp
from jax import lax
from jax.experimental import pallas as pl
from jax.experimental.pallas import tpu as pltpu
from jax.experimental.pallas import tpu_sc as plsc

N_NODES = 10000
N_EDGES = 320000
D = 128
DH = D // 2
NC = 2
NS = 16
EPW = N_EDGES // NS
CHUNK = 80
NCHUNK = EPW // CHUNK
HIST_SPLIT = 120
GROUP = 10
CNT_W = 16
NP = 10240
ROWS_PER_SUB = NP // NS
CNT_ROWS = 2 * NP
CNT_PER_SUB = CNT_ROWS // NS
VPC = CHUNK // 16


def _sc_aggregate(x_rows, src1d, dst1d, zeros_agg, zeros_cnt, ones_rows):
    mesh = plsc.VectorSubcoreMesh(core_axis_name="c", subcore_axis_name="s")

    @functools.partial(
        pl.kernel,
        out_type=(
            jax.ShapeDtypeStruct((NP, D), jnp.float32),
            jax.ShapeDtypeStruct((NC * CNT_ROWS, CNT_W), jnp.float32),
        ),
        mesh=mesh,
        scratch_types=[
            pltpu.VMEM((2, GROUP * CHUNK), jnp.int32),
            pltpu.VMEM((2, GROUP * CHUNK), jnp.int32),
            pltpu.VMEM((GROUP, CHUNK), jnp.int32),
            pltpu.VMEM((GROUP, CHUNK), jnp.int32),
            pltpu.VMEM((GROUP, CHUNK, DH), jnp.float32),
            pltpu.VMEM((CHUNK, CNT_W), jnp.float32),
            pltpu.VMEM_SHARED((NP, DH), jnp.float32),
            pltpu.VMEM_SHARED((CNT_ROWS, CNT_W), jnp.float32),
        ]
        + [pltpu.SemaphoreType.DMA] * GROUP
        + [pltpu.SemaphoreType.DMA,
           pltpu.SemaphoreType.DMA,
           pltpu.SemaphoreType.DMA,
           pltpu.SemaphoreType.DMA],
        compiler_params=pltpu.CompilerParams(use_tc_tiling_on_sc=False),
    )
    def k(x_hbm, src_hbm, dst_hbm, zagg_hbm, zcnt_hbm, ones_hbm,
          agg_out, cnt_out,
          sraw, draw, sidx, didx, rows, ones_v, agg_sh, cnt_sh, *sems):
        gsems = sems[:GROUP]
        ssem = sems[GROUP]
        hsem = sems[GROUP + 1]
        isems = (sems[GROUP + 2], sems[GROUP + 3])
        cid = lax.axis_index("c")
        sid = lax.axis_index("s")

        rbase = sid * ROWS_PER_SUB
        cbase = sid * CNT_PER_SUB
        init = [pltpu.async_copy(ones_hbm, ones_v, gsems[0])]
        pltpu.sync_copy(zagg_hbm.at[pl.ds(rbase, ROWS_PER_SUB)],
                        agg_sh.at[pl.ds(rbase, ROWS_PER_SUB)])
        pltpu.sync_copy(zcnt_hbm.at[pl.ds(cbase, CNT_PER_SUB)],
                        cnt_sh.at[pl.ds(cbase, CNT_PER_SUB)])
        for cp in init:
            cp.wait()

        def load_block(j, pb):
            base = sid * EPW + j * CHUNK
            return [
                pltpu.async_copy(src_hbm.at[pl.ds(base, GROUP * CHUNK)],
                                 sraw.at[pb], isems[pb]),
                pltpu.async_copy(dst_hbm.at[pl.ds(base, GROUP * CHUNK)],
                                 draw.at[pb], isems[pb]),
            ]

        def do_group(j, pb):
            base = sid * EPW + j * CHUNK
            pltpu.make_async_copy(src_hbm.at[pl.ds(base, GROUP * CHUNK)],
                                  sraw.at[pb], isems[pb]).wait()
            pltpu.make_async_copy(dst_hbm.at[pl.ds(base, GROUP * CHUNK)],
                                  draw.at[pb], isems[pb]).wait()

            @pl.when(j + GROUP < NCHUNK)
            def _():
                load_block(j + GROUP, 1 - pb)

            for b in range(GROUP):
                for v in range(VPC):
                    fl = (pb, pl.ds(b * CHUNK + v * 16, 16))
                    sl = (b, pl.ds(v * 16, 16))
                    iv = sraw[fl]
                    sidx[sl] = iv + iv + cid
                    didx[sl] = draw[fl]

            gcps = [pltpu.async_copy(
                x_hbm.at[sidx.at[b]], rows.at[b], gsems[b])
                for b in range(GROUP)]

            @pl.when((j < HIST_SPLIT) == (cid == 0))
            def _():
                hcps = [pltpu.async_copy(
                    ones_v, cnt_sh.at[sidx.at[b]], hsem, add=True)
                    for b in range(GROUP)]
                for cp in hcps:
                    cp.wait()

            scps = []
            for b in range(GROUP):
                gcps[b].wait()
                scps.append(pltpu.async_copy(
                    rows.at[b], agg_sh.at[didx.at[b]], ssem, add=True))
            for cp in scps:
                cp.wait()

        load_block(0, 0)
        plsc.subcore_barrier()

        @pl.loop(0, NCHUNK - GROUP, step=2 * GROUP)
        def _(j):
            do_group(j, 0)
            do_group(j + GROUP, 1)

        do_group(NCHUNK - GROUP, 0)
        plsc.subcore_barrier()
        pltpu.sync_copy(
            agg_sh.at[pl.ds(rbase, ROWS_PER_SUB)],
            agg_out.at[pl.ds(rbase, ROWS_PER_SUB), pl.ds(cid * DH, DH)])
        pltpu.sync_copy(cnt_sh.at[pl.ds(cbase, CNT_PER_SUB)],
                        cnt_out.at[pl.ds(cid * CNT_ROWS + cbase, CNT_PER_SUB)])

    return k(x_rows, src1d, dst1d, zeros_agg, zeros_cnt, ones_rows)


def _tc_dense_body(x_ref, agg_ref, cnt_ref, w1_ref, b1_ref, w2_ref, b2_ref,
                   out_ref):
    h = x_ref[...] + agg_ref[:N_NODES, :]
    z = jnp.dot(h, w1_ref[...], preferred_element_type=jnp.float32) + b1_ref[...]
    r = jnp.maximum(z, 0.0)
    w = 1.0 + cnt_ref[:N_NODES, 0:1] + cnt_ref[NP:NP + N_NODES, 16:17]
    s = jnp.sum(r * w, axis=0, keepdims=True)
    out_ref[...] = (jnp.dot(s, w2_ref[...], preferred_element_type=jnp.float32)
                    + float(N_NODES) * b2_ref[...])


def _tc_dense(feats, agg, cnt32, W1, b1, W2, b2):
    return pl.pallas_call(
        _tc_dense_body,
        out_shape=jax.ShapeDtypeStruct((1, D), jnp.float32),
    )(feats, agg, cnt32, W1, b1.reshape(1, D), W2, b2.reshape(1, D))


def kernel(feats, edge_index, W1, b1, W2, b2):
    ei = edge_index.astype(jnp.int32)
    src1d = ei[0]
    dst1d = ei[1]
    x_rows = feats.reshape(2 * N_NODES, DH)
    zeros_agg = jnp.zeros((NP, DH), jnp.float32)
    zeros_cnt = jnp.zeros((CNT_ROWS, CNT_W), jnp.float32)
    ones_rows = jnp.zeros((CHUNK, CNT_W), jnp.float32).at[:, 0].set(1.0)
    agg, cnt = _sc_aggregate(x_rows, src1d, dst1d, zeros_agg, zeros_cnt,
                             ones_rows)
    cnt32 = cnt.reshape(NC * NP, 2 * CNT_W)
    return _tc_dense(feats, agg, cnt32, W1, b1, W2, b2)


# speedup vs baseline: 1.0042x; 1.0042x over previous
import functools

import jax
import jax.numpy as jnp
from jax import lax
from jax.experimental import pallas as pl
from jax.experimental.pallas import tpu as pltpu
from jax.experimental.pallas import tpu_sc as plsc

N_NODES = 10000
N_EDGES = 320000
D = 128
DH = D // 2
NC = 2
NS = 16
EPW = N_EDGES // NS
CHUNK = 80
NCHUNK = EPW // CHUNK
HIST_SPLIT = 120
GROUP = 10
CNT_W = 16
NP = 10240
ROWS_PER_SUB = NP // NS
CNT_ROWS = NP
CNT_PER_SUB = CNT_ROWS // NS
VPC = CHUNK // 16


def _sc_aggregate(x_rows, src1d, dst1d, zeros_agg, zeros_cnt, ones_rows):
    mesh = plsc.VectorSubcoreMesh(core_axis_name="c", subcore_axis_name="s")

    @functools.partial(
        pl.kernel,
        out_type=(
            jax.ShapeDtypeStruct((NP, D), jnp.float32),
            jax.ShapeDtypeStruct((NC * CNT_ROWS, CNT_W), jnp.float32),
        ),
        mesh=mesh,
        scratch_types=[
            pltpu.VMEM((2, GROUP * CHUNK), jnp.int32),
            pltpu.VMEM((2, GROUP * CHUNK), jnp.int32),
            pltpu.VMEM((GROUP, CHUNK), jnp.int32),
            pltpu.VMEM((GROUP, CHUNK), jnp.int32),
            pltpu.VMEM((GROUP, CHUNK), jnp.int32),
            pltpu.VMEM((GROUP, CHUNK, DH), jnp.float32),
            pltpu.VMEM((CHUNK, CNT_W), jnp.float32),
            pltpu.VMEM_SHARED((NP, DH), jnp.float32),
            pltpu.VMEM_SHARED((CNT_ROWS, CNT_W), jnp.float32),
        ]
        + [pltpu.SemaphoreType.DMA] * GROUP
        + [pltpu.SemaphoreType.DMA,
           pltpu.SemaphoreType.DMA,
           pltpu.SemaphoreType.DMA,
           pltpu.SemaphoreType.DMA],
        compiler_params=pltpu.CompilerParams(use_tc_tiling_on_sc=False),
    )
    def k(x_hbm, src_hbm, dst_hbm, zagg_hbm, zcnt_hbm, ones_hbm,
          agg_out, cnt_out,
          sraw, draw, sidx, hidx, didx, rows, ones_v, agg_sh, cnt_sh, *sems):
        gsems = sems[:GROUP]
        ssem = sems[GROUP]
        hsem = sems[GROUP + 1]
        isems = (sems[GROUP + 2], sems[GROUP + 3])
        cid = lax.axis_index("c")
        sid = lax.axis_index("s")

        rbase = sid * ROWS_PER_SUB
        cbase = sid * CNT_PER_SUB
        init = [pltpu.async_copy(ones_hbm, ones_v, gsems[0])]
        pltpu.sync_copy(zagg_hbm.at[pl.ds(rbase, ROWS_PER_SUB)],
                        agg_sh.at[pl.ds(rbase, ROWS_PER_SUB)])
        pltpu.sync_copy(zcnt_hbm.at[pl.ds(cbase, CNT_PER_SUB)],
                        cnt_sh.at[pl.ds(cbase, CNT_PER_SUB)])
        for cp in init:
            cp.wait()

        def load_block(j, pb):
            base = sid * EPW + j * CHUNK
            return [
                pltpu.async_copy(src_hbm.at[pl.ds(base, GROUP * CHUNK)],
                                 sraw.at[pb], isems[pb]),
                pltpu.async_copy(dst_hbm.at[pl.ds(base, GROUP * CHUNK)],
                                 draw.at[pb], isems[pb]),
            ]

        def do_group(j, pb):
            base = sid * EPW + j * CHUNK
            pltpu.make_async_copy(src_hbm.at[pl.ds(base, GROUP * CHUNK)],
                                  sraw.at[pb], isems[pb]).wait()
            pltpu.make_async_copy(dst_hbm.at[pl.ds(base, GROUP * CHUNK)],
                                  draw.at[pb], isems[pb]).wait()

            @pl.when(j + GROUP < NCHUNK)
            def _():
                load_block(j + GROUP, 1 - pb)

            for b in range(GROUP):
                for v in range(VPC):
                    fl = (pb, pl.ds(b * CHUNK + v * 16, 16))
                    sl = (b, pl.ds(v * 16, 16))
                    iv = sraw[fl]
                    sidx[sl] = iv + iv + cid
                    hidx[sl] = iv
                    didx[sl] = draw[fl]

            gcps = [pltpu.async_copy(
                x_hbm.at[sidx.at[b]], rows.at[b], gsems[b])
                for b in range(GROUP)]

            @pl.when((j < HIST_SPLIT) == (cid == 0))
            def _():
                hcps = [pltpu.async_copy(
                    ones_v, cnt_sh.at[hidx.at[b]], hsem, add=True)
                    for b in range(GROUP)]
                for cp in hcps:
                    cp.wait()

            scps = []
            for b in range(GROUP):
                gcps[b].wait()
                scps.append(pltpu.async_copy(
                    rows.at[b], agg_sh.at[didx.at[b]], ssem, add=True))
            for cp in scps:
                cp.wait()

        load_block(0, 0)
        plsc.subcore_barrier()

        @pl.loop(0, NCHUNK - GROUP, step=2 * GROUP)
        def _(j):
            do_group(j, 0)
            do_group(j + GROUP, 1)

        do_group(NCHUNK - GROUP, 0)
        plsc.subcore_barrier()
        pltpu.sync_copy(
            agg_sh.at[pl.ds(rbase, ROWS_PER_SUB)],
            agg_out.at[pl.ds(rbase, ROWS_PER_SUB), pl.ds(cid * DH, DH)])
        pltpu.sync_copy(cnt_sh.at[pl.ds(cbase, CNT_PER_SUB)],
                        cnt_out.at[pl.ds(cid * CNT_ROWS + cbase, CNT_PER_SUB)])

    return k(x_rows, src1d, dst1d, zeros_agg, zeros_cnt, ones_rows)


def _tc_dense_body(x_ref, agg_ref, w_ref, w1_ref, b1_ref, w2_ref, b2_ref,
                   out_ref):
    h = x_ref[...] + agg_ref[:N_NODES, :]
    z = jnp.dot(h, w1_ref[...], preferred_element_type=jnp.float32) + b1_ref[...]
    r = jnp.maximum(z, 0.0)
    s = jnp.sum(r * w_ref[...], axis=0, keepdims=True)
    out_ref[...] = (jnp.dot(s, w2_ref[...], preferred_element_type=jnp.float32)
                    + float(N_NODES) * b2_ref[...])


def _tc_dense(feats, agg, w, W1, b1, W2, b2):
    return pl.pallas_call(
        _tc_dense_body,
        out_shape=jax.ShapeDtypeStruct((1, D), jnp.float32),
    )(feats, agg, w, W1, b1.reshape(1, D), W2, b2.reshape(1, D))


def kernel(feats, edge_index, W1, b1, W2, b2):
    ei = edge_index.astype(jnp.int32)
    src1d = ei[0]
    dst1d = ei[1]
    x_rows = feats.reshape(2 * N_NODES, DH)
    zeros_agg = jnp.zeros((NP, DH), jnp.float32)
    zeros_cnt = jnp.zeros((CNT_ROWS, CNT_W), jnp.float32)
    ones_rows = jnp.zeros((CHUNK, CNT_W), jnp.float32).at[:, 0].set(1.0)
    agg, cnt = _sc_aggregate(x_rows, src1d, dst1d, zeros_agg, zeros_cnt,
                             ones_rows)
    w = (1.0 + cnt[:N_NODES, 0] + cnt[NP:NP + N_NODES, 0]).reshape(N_NODES, 1)
    return _tc_dense(feats, agg, w, W1, b1, W2, b2)
